# baseline (speedup 1.0000x reference)
import functools

import jax
import jax.numpy as jnp
from jax import lax
from jax.experimental import pallas as pl
from jax.experimental.pallas import tpu as pltpu
from jax.experimental.pallas import tpu_sc as plsc

B = 16
D = 32
HID = 128
HH = 512
WW = 512
POS = HH * WW
NCORE = 2
NSUB = 16
NW = NCORE * NSUB
ROWS128 = POS * D // 128
RPW = ROWS128 // NCORE
CH_R = 128
TOTCH = RPW // CH_R
NBUF = 4
RING = TOTCH // NBUF
UNROLL = 4


def _tree_sum(vs):
    while len(vs) > 1:
        nxt = [vs[i] + vs[i + 1] for i in range(0, len(vs) - 1, 2)]
        if len(vs) % 2:
            nxt.append(vs[-1])
        vs = nxt
    return vs[0]


def _sc_reduce_body(x_hbm, out_hbm, buf, accv, sem):
    b = lax.axis_index("s")
    half = lax.axis_index("c")
    r0 = half * RPW

    def src(chunk):
        return x_hbm.at[b, pl.ds(r0 + chunk * CH_R, CH_R), :]

    def start(chunk, slot):
        pltpu.make_async_copy(src(chunk), buf.at[slot], sem.at[slot]).start()

    for s in range(NBUF):
        start(s, s)

    zero = jnp.zeros((16,), jnp.float32)

    def outer(i, accs):
        new = list(accs)
        for s in range(NBUF):
            chunk = i * NBUF + s
            pltpu.make_async_copy(src(chunk), buf.at[s], sem.at[s]).wait()

            def inner(j, c2, s=s):
                lo, hi = c2
                base = j * UNROLL
                lo_loads = [buf[s, base + u, q * 32:q * 32 + 16]
                            for u in range(UNROLL) for q in range(4)]
                hi_loads = [buf[s, base + u, q * 32 + 16:q * 32 + 32]
                            for u in range(UNROLL) for q in range(4)]
                return (lo + _tree_sum(lo_loads), hi + _tree_sum(hi_loads))

            lo, hi = lax.fori_loop(0, CH_R // UNROLL, inner,
                                   (new[2 * s], new[2 * s + 1]))
            new[2 * s], new[2 * s + 1] = lo, hi

            @pl.when(chunk + NBUF < TOTCH)
            def _(chunk=chunk, s=s):
                start(chunk + NBUF, s)

        return tuple(new)

    accs = lax.fori_loop(0, RING, outer, (zero,) * (2 * NBUF))
    accv[0, 0:16] = _tree_sum([accs[2 * s] for s in range(NBUF)])
    accv[1, 0:16] = _tree_sum([accs[2 * s + 1] for s in range(NBUF)])
    w = b * NCORE + half
    pltpu.make_async_copy(accv, out_hbm.at[w], sem.at[NBUF]).start()
    pltpu.make_async_copy(accv, out_hbm.at[w], sem.at[NBUF]).wait()


_sc_reduce = functools.partial(
    pl.kernel,
    out_type=jax.ShapeDtypeStruct((NW, 2, 16), jnp.float32),
    mesh=plsc.VectorSubcoreMesh(core_axis_name="c", subcore_axis_name="s"),
    compiler_params=pltpu.CompilerParams(use_tc_tiling_on_sc=False),
    scratch_types=[
        pltpu.VMEM((NBUF, CH_R, 128), jnp.float32),
        pltpu.VMEM((2, 16), jnp.float32),
        pltpu.SemaphoreType.DMA((NBUF + 1,)),
    ],
)(_sc_reduce_body)


def _head_body(p_ref, w1_ref, b1_ref, w2_ref, b2_ref, g_ref,
               rout_ref, probs_ref, idx_ref):
    x = jnp.sum(p_ref[...], axis=1) * (1.0 / POS)
    h = jnp.maximum(
        jax.lax.dot_general(x, w1_ref[...], (((1,), (0,)), ((), ())),
                            preferred_element_type=jnp.float32) + b1_ref[...],
        0.0)
    logits = jax.lax.dot_general(h, w2_ref[...], (((1,), (0,)), ((), ())),
                                 preferred_element_type=jnp.float32) + b2_ref[...]
    m = jnp.max(logits, axis=-1, keepdims=True)
    e = jnp.exp(logits - m)
    probs = e / jnp.sum(e, axis=-1, keepdims=True)
    probs_ref[...] = probs
    z = jnp.log(probs + 1e-20) + g_ref[...]
    zmax = jnp.max(z, axis=-1, keepdims=True)
    lane = jax.lax.broadcasted_iota(jnp.int32, (B, D), 1)
    idx = jnp.min(jnp.where(z >= zmax, lane, D), axis=-1, keepdims=True)
    idx_ref[...] = idx
    rout_ref[...] = (lane == idx).astype(jnp.float32)


@functools.partial(jax.jit, static_argnames=())
def kernel(attn_5d, W1, b1, W2, b2):
    x3 = attn_5d.reshape(B, ROWS128, 128)
    partial = _sc_reduce(x3)
    partial = partial.reshape(B, 2, D)

    gumbel = jax.random.gumbel(jax.random.key(1234), (B, D), jnp.float32)
    rout, probs, idx = pl.pallas_call(
        _head_body,
        in_specs=[
            pl.BlockSpec((B, 2, D), lambda: (0, 0, 0)),
            pl.BlockSpec((D, HID), lambda: (0, 0)),
            pl.BlockSpec((1, HID), lambda: (0, 0)),
            pl.BlockSpec((HID, D), lambda: (0, 0)),
            pl.BlockSpec((1, D), lambda: (0, 0)),
            pl.BlockSpec((B, D), lambda: (0, 0)),
        ],
        out_specs=[
            pl.BlockSpec((B, D), lambda: (0, 0)),
            pl.BlockSpec((B, D), lambda: (0, 0)),
            pl.BlockSpec((B, 1), lambda: (0, 0)),
        ],
        out_shape=[
            jax.ShapeDtypeStruct((B, D), jnp.float32),
            jax.ShapeDtypeStruct((B, D), jnp.float32),
            jax.ShapeDtypeStruct((B, 1), jnp.int32),
        ],
    )(partial, W1, b1.reshape(1, HID), W2, b2.reshape(1, D), gumbel)
    return rout, probs, idx.reshape(B)


# speedup vs baseline: 1.8886x; 1.8886x over previous
import functools

import jax
import jax.numpy as jnp
from jax.experimental import pallas as pl
from jax.experimental.pallas import tpu as pltpu

B = 16
D = 32
HID = 128
HH = 512
POS = 512 * 512
ROWLEN = 512 * D
BH = 64
NSTEPS = HH // BH
ACC = 64


def _reduce_body(x_ref, acc_ref):
    j = pl.program_id(1)

    @pl.when(j == 0)
    def _():
        acc_ref[...] = jnp.zeros_like(acc_ref)

    x = x_ref[0]
    acc_ref[0] += jnp.sum(x.reshape(BH * ROWLEN // (ACC * 128), ACC, 128), axis=0)


def _head_body(p_ref, w1_ref, b1_ref, w2_ref, b2_ref, g_ref,
               rout_ref, probs_ref, idx_ref):
    p = jnp.sum(p_ref[...], axis=1)
    x = (p[:, 0:32] + p[:, 32:64] + p[:, 64:96] + p[:, 96:128]) * (1.0 / POS)
    h = jnp.maximum(
        jax.lax.dot_general(x, w1_ref[...], (((1,), (0,)), ((), ())),
                            preferred_element_type=jnp.float32) + b1_ref[...],
        0.0)
    logits = jax.lax.dot_general(h, w2_ref[...], (((1,), (0,)), ((), ())),
                                 preferred_element_type=jnp.float32) + b2_ref[...]
    m = jnp.max(logits, axis=-1, keepdims=True)
    e = jnp.exp(logits - m)
    probs = e / jnp.sum(e, axis=-1, keepdims=True)
    probs_ref[...] = probs
    z = jnp.log(probs + 1e-20) + g_ref[...]
    zmax = jnp.max(z, axis=-1, keepdims=True)
    lane = jax.lax.broadcasted_iota(jnp.int32, (B, D), 1)
    idx = jnp.min(jnp.where(z >= zmax, lane, D), axis=-1, keepdims=True)
    idx_ref[...] = idx
    rout_ref[...] = (lane == idx).astype(jnp.float32)


@functools.partial(jax.jit, static_argnames=())
def kernel(attn_5d, W1, b1, W2, b2):
    x = attn_5d.reshape(B, HH, ROWLEN)
    partial = pl.pallas_call(
        _reduce_body,
        grid=(B, NSTEPS),
        in_specs=[pl.BlockSpec((1, BH, ROWLEN), lambda b, j: (b, j, 0))],
        out_specs=pl.BlockSpec((1, ACC, 128), lambda b, j: (b, 0, 0)),
        out_shape=jax.ShapeDtypeStruct((B, ACC, 128), jnp.float32),
    )(x)

    gumbel = jax.random.gumbel(jax.random.key(1234), (B, D), jnp.float32)
    rout, probs, idx = pl.pallas_call(
        _head_body,
        in_specs=[
            pl.BlockSpec((B, ACC, 128), lambda: (0, 0, 0)),
            pl.BlockSpec((D, HID), lambda: (0, 0)),
            pl.BlockSpec((1, HID), lambda: (0, 0)),
            pl.BlockSpec((HID, D), lambda: (0, 0)),
            pl.BlockSpec((1, D), lambda: (0, 0)),
            pl.BlockSpec((B, D), lambda: (0, 0)),
        ],
        out_specs=[
            pl.BlockSpec((B, D), lambda: (0, 0)),
            pl.BlockSpec((B, D), lambda: (0, 0)),
            pl.BlockSpec((B, 1), lambda: (0, 0)),
        ],
        out_shape=[
            jax.ShapeDtypeStruct((B, D), jnp.float32),
            jax.ShapeDtypeStruct((B, D), jnp.float32),
            jax.ShapeDtypeStruct((B, 1), jnp.int32),
        ],
    )(partial, W1, b1.reshape(1, HID), W2, b2.reshape(1, D), gumbel)
    return rout, probs, idx.reshape(B)


# speedup vs baseline: 1.8892x; 1.0003x over previous
import functools

import jax
import jax.numpy as jnp
from jax.experimental import pallas as pl

B = 16
D = 32
HID = 128
HH = 512
POS = 512 * 512
ROWLEN = 512 * D
BH = 64
NSTEPS = HH // BH
ACC = 64


def _reduce_body(x_ref, acc_ref):
    j = pl.program_id(1)

    @pl.when(j == 0)
    def _():
        acc_ref[...] = jnp.zeros_like(acc_ref)

    x = x_ref[0]
    acc_ref[0] += jnp.sum(x.reshape(BH * ROWLEN // (ACC * 128), ACC, 128), axis=0)


def _head_body(p_ref, w1_ref, b1_ref, w2_ref, b2_ref, g_ref,
               rout_ref, probs_ref, idx_ref):
    p = jnp.sum(p_ref[...], axis=1)
    x = (p[:, 0:32] + p[:, 32:64] + p[:, 64:96] + p[:, 96:128]) * (1.0 / POS)
    h = jnp.maximum(
        jax.lax.dot_general(x, w1_ref[...], (((1,), (0,)), ((), ())),
                            preferred_element_type=jnp.float32) + b1_ref[...],
        0.0)
    logits = jax.lax.dot_general(h, w2_ref[...], (((1,), (0,)), ((), ())),
                                 preferred_element_type=jnp.float32) + b2_ref[...]
    m = jnp.max(logits, axis=-1, keepdims=True)
    e = jnp.exp(logits - m)
    probs = e / jnp.sum(e, axis=-1, keepdims=True)
    probs_ref[...] = probs
    z = jnp.log(probs + 1e-20) + g_ref[...]
    zmax = jnp.max(z, axis=-1, keepdims=True)
    lane = jax.lax.broadcasted_iota(jnp.int32, (B, D), 1)
    idx = jnp.min(jnp.where(z >= zmax, lane, D), axis=-1, keepdims=True)
    idx_ref[...] = idx
    rout_ref[...] = (lane == idx).astype(jnp.float32)


@functools.partial(jax.jit, static_argnames=())
def kernel(attn_5d, W1, b1, W2, b2):
    x = attn_5d.reshape(B, HH, ROWLEN)
    partial = pl.pallas_call(
        _reduce_body,
        grid=(B, NSTEPS),
        in_specs=[pl.BlockSpec((1, BH, ROWLEN), lambda b, j: (b, j, 0))],
        out_specs=pl.BlockSpec((1, ACC, 128), lambda b, j: (b, 0, 0)),
        out_shape=jax.ShapeDtypeStruct((B, ACC, 128), jnp.float32),
    )(x)

    gumbel = jax.random.gumbel(jax.random.key(1234), (B, D), jnp.float32)
    rout, probs, idx = pl.pallas_call(
        _head_body,
        in_specs=[
            pl.BlockSpec((B, ACC, 128), lambda: (0, 0, 0)),
            pl.BlockSpec((D, HID), lambda: (0, 0)),
            pl.BlockSpec((1, HID), lambda: (0, 0)),
            pl.BlockSpec((HID, D), lambda: (0, 0)),
            pl.BlockSpec((1, D), lambda: (0, 0)),
            pl.BlockSpec((B, D), lambda: (0, 0)),
        ],
        out_specs=[
            pl.BlockSpec((B, D), lambda: (0, 0)),
            pl.BlockSpec((B, D), lambda: (0, 0)),
            pl.BlockSpec((B, 1), lambda: (0, 0)),
        ],
        out_shape=[
            jax.ShapeDtypeStruct((B, D), jnp.float32),
            jax.ShapeDtypeStruct((B, D), jnp.float32),
            jax.ShapeDtypeStruct((B, 1), jnp.int32),
        ],
    )(partial, W1, b1.reshape(1, HID), W2, b2.reshape(1, D), gumbel)
    return rout, probs, idx.reshape(B)
